# baseline (speedup 1.0000x reference)
import functools

import jax
import jax.numpy as jnp
from jax import lax
from jax.experimental import pallas as pl
from jax.experimental.pallas import tpu as pltpu


def _fps_body(m, x_ref, y_ref, z_ref, out_ref):
    R = x_ref.shape[0]
    x = x_ref[:, :]
    y = y_ref[:, :]
    z = z_ref[:, :]
    rows_out = m // 128
    iota_flat = (lax.broadcasted_iota(jnp.int32, (R, 128), 0) * 128
                 + lax.broadcasted_iota(jnp.int32, (R, 128), 1))
    lane = lax.broadcasted_iota(jnp.int32, (1, 128), 1)
    out_iota = (lax.broadcasted_iota(jnp.int32, (rows_out, 128), 0) * 128
                + lax.broadcasted_iota(jnp.int32, (rows_out, 128), 1))

    def body(i, state):
        dists, idxs, r, c = state
        idxs = jnp.where(out_iota == i, r * 128 + c, idxs)
        cm = lane == c
        xf = jnp.sum(jnp.where(cm, x_ref[pl.ds(r, 1), :], 0.0))
        yf = jnp.sum(jnp.where(cm, y_ref[pl.ds(r, 1), :], 0.0))
        zf = jnp.sum(jnp.where(cm, z_ref[pl.ds(r, 1), :], 0.0))
        d = (x - xf) ** 2 + (y - yf) ** 2 + (z - zf) ** 2
        dists = jnp.minimum(dists, d)
        mx = jnp.max(dists)
        far = jnp.min(jnp.where(dists == mx, iota_flat, jnp.int32(R * 128)))
        return dists, idxs, far // 128, far % 128

    dists0 = jnp.full((R, 128), 1e10, dtype=jnp.float32)
    idxs0 = jnp.zeros((rows_out, 128), dtype=jnp.int32)
    _, idxs, _, _ = lax.fori_loop(
        0, m, body, (dists0, idxs0, jnp.int32(0), jnp.int32(0)))
    out_ref[:, :] = idxs


def _fps(pos, m):
    n = pos.shape[0]
    x = pos[:, 0].reshape(n // 128, 128)
    y = pos[:, 1].reshape(n // 128, 128)
    z = pos[:, 2].reshape(n // 128, 128)
    idxs = pl.pallas_call(
        functools.partial(_fps_body, m),
        out_shape=jax.ShapeDtypeStruct((m // 128, 128), jnp.int32),
    )(x, y, z)
    return idxs.reshape(m)


def _knn(query, ref, k):
    d = (jnp.sum(query * query, axis=1, keepdims=True)
         + jnp.sum(ref * ref, axis=1)[None, :]
         - 2.0 * (query @ ref.T))
    neg_d, idx = lax.top_k(-d, k)
    return idx, -neg_d


def _bn(x):
    axes = tuple(range(x.ndim - 1))
    mean = jnp.mean(x, axis=axes, keepdims=True)
    var = jnp.var(x, axis=axes, keepdims=True)
    return (x - mean) / jnp.sqrt(var + 1e-5)


def _mlp(x, layers):
    for W, b in layers:
        x = jax.nn.relu(_bn(x @ W + b))
    return x


def kernel(pos, feat, offset, params):
    del offset
    feat0 = jnp.concatenate([pos, feat], axis=1)

    o = _fps(pos, 4096)
    pos_l = [pos, pos[o[:4096]], pos[o[:1024]], pos[o[:256]], pos[o[:64]]]

    feats = [feat0]
    for lvl, (name, nsub) in enumerate(
            [('sa1', 4096), ('sa2', 1024), ('sa3', 256), ('sa4', 64)]):
        p_in, p_out = pos_l[lvl], pos_l[lvl + 1]
        nn_idx, _ = _knn(p_out, p_in, 32)
        grouped_pos = p_in[nn_idx] - p_out[:, None, :]
        grouped = jnp.concatenate([grouped_pos, feats[-1][nn_idx]], axis=-1)
        feats.append(jnp.max(_mlp(grouped, params[name]), axis=1))

    f1, f2, f3, f4 = feats[1], feats[2], feats[3], feats[4]

    def fp(pos1, feat1, pos2, feat2, layers):
        idx, d = _knn(pos1, pos2, 3)
        w = 1.0 / (d + 1e-8)
        w = w / jnp.sum(w, axis=1, keepdims=True)
        interp = jnp.sum(feat2[idx] * w[..., None], axis=1)
        x = interp if feat1 is None else jnp.concatenate([feat1, interp], axis=-1)
        return _mlp(x, layers)

    f3 = fp(pos_l[3], f3, pos_l[4], f4, params['fp4'])
    f2 = fp(pos_l[2], f2, pos_l[3], f3, params['fp3'])
    f1 = fp(pos_l[1], f1, pos_l[2], f2, params['fp2'])
    x = fp(pos_l[0], None, pos_l[1], f1, params['fp1'])
    (w1, b1), (w2, b2) = params['cls']
    x = jax.nn.relu(_bn(x @ w1 + b1))
    x = x @ w2 + b2
    return x


# speedup vs baseline: 1.0122x; 1.0122x over previous
import functools

import jax
import jax.numpy as jnp
from jax import lax
from jax.experimental import pallas as pl
from jax.experimental.pallas import tpu as pltpu
from jax.experimental.pallas import tpu_sc as plsc


def _fps_body(m, x_ref, y_ref, z_ref, out_ref):
    R = x_ref.shape[0]
    x = x_ref[:, :]
    y = y_ref[:, :]
    z = z_ref[:, :]
    rows_out = m // 128
    iota_flat = (lax.broadcasted_iota(jnp.int32, (R, 128), 0) * 128
                 + lax.broadcasted_iota(jnp.int32, (R, 128), 1))
    lane = lax.broadcasted_iota(jnp.int32, (1, 128), 1)
    out_iota = (lax.broadcasted_iota(jnp.int32, (rows_out, 128), 0) * 128
                + lax.broadcasted_iota(jnp.int32, (rows_out, 128), 1))

    def body(i, state):
        dists, idxs, r, c = state
        idxs = jnp.where(out_iota == i, r * 128 + c, idxs)
        cm = lane == c
        xf = jnp.sum(jnp.where(cm, x_ref[pl.ds(r, 1), :], 0.0))
        yf = jnp.sum(jnp.where(cm, y_ref[pl.ds(r, 1), :], 0.0))
        zf = jnp.sum(jnp.where(cm, z_ref[pl.ds(r, 1), :], 0.0))
        d = (x - xf) ** 2 + (y - yf) ** 2 + (z - zf) ** 2
        dists = jnp.minimum(dists, d)
        mx = jnp.max(dists)
        far = jnp.min(jnp.where(dists == mx, iota_flat, jnp.int32(R * 128)))
        return dists, idxs, far // 128, far % 128

    dists0 = jnp.full((R, 128), 1e10, dtype=jnp.float32)
    idxs0 = jnp.zeros((rows_out, 128), dtype=jnp.int32)
    _, idxs, _, _ = lax.fori_loop(
        0, m, body, (dists0, idxs0, jnp.int32(0), jnp.int32(0)))
    out_ref[:, :] = idxs


def _fps(pos, m):
    n = pos.shape[0]
    x = pos[:, 0].reshape(n // 128, 128)
    y = pos[:, 1].reshape(n // 128, 128)
    z = pos[:, 2].reshape(n // 128, 128)
    idxs = pl.pallas_call(
        functools.partial(_fps_body, m),
        out_shape=jax.ShapeDtypeStruct((m // 128, 128), jnp.int32),
    )(x, y, z)
    return idxs.reshape(m)


def _knn(query, ref, k):
    d = (jnp.sum(query * query, axis=1, keepdims=True)
         + jnp.sum(ref * ref, axis=1)[None, :]
         - 2.0 * (query @ ref.T))
    neg_d, idx = lax.top_k(-d, k)
    return idx, -neg_d


def _sc_gather(table, idx):
    V, D = table.shape
    B = idx.shape[0]
    NW = 32
    b_per_w = B // NW
    CH = 512 if b_per_w >= 512 else b_per_w
    nch = b_per_w // CH
    mesh = plsc.VectorSubcoreMesh(core_axis_name="c", subcore_axis_name="s")

    @functools.partial(
        pl.kernel, mesh=mesh,
        out_type=jax.ShapeDtypeStruct((B, D), jnp.float32),
        scratch_types=[
            pltpu.VMEM((CH,), jnp.int32),
            pltpu.VMEM((CH, D), jnp.float32),
            pltpu.SemaphoreType.DMA,
        ],
    )
    def k(table_hbm, idx_hbm, out_hbm, idx_v, rows_v, sem):
        wid = lax.axis_index("s") * 2 + lax.axis_index("c")
        base = wid * b_per_w

        def body(ch, _):
            off = base + ch * CH
            pltpu.sync_copy(idx_hbm.at[pl.ds(off, CH)], idx_v)
            pltpu.async_copy(table_hbm.at[idx_v], rows_v, sem).wait()
            pltpu.sync_copy(rows_v, out_hbm.at[pl.ds(off, CH)])
            return 0

        lax.fori_loop(0, nch, body, 0)

    return k(table, idx)


def _gather_rows(table, idx):
    V, D = table.shape
    flat = idx.reshape(-1)
    B = flat.shape[0]
    if D % 16 == 0 and B % 256 == 0 and (B // 32) % 8 == 0:
        out = _sc_gather(table, flat)
    else:
        out = table[flat]
    return out.reshape(idx.shape + (D,))


def _bn(x):
    axes = tuple(range(x.ndim - 1))
    mean = jnp.mean(x, axis=axes, keepdims=True)
    var = jnp.var(x, axis=axes, keepdims=True)
    return (x - mean) / jnp.sqrt(var + 1e-5)


def _mlp(x, layers):
    for W, b in layers:
        x = jax.nn.relu(_bn(x @ W + b))
    return x


def kernel(pos, feat, offset, params):
    del offset
    feat0 = jnp.concatenate([pos, feat], axis=1)

    o = _fps(pos, 4096)
    pos_l = [pos, pos[o[:4096]], pos[o[:1024]], pos[o[:256]], pos[o[:64]]]

    feats = [feat0]
    for lvl, (name, nsub) in enumerate(
            [('sa1', 4096), ('sa2', 1024), ('sa3', 256), ('sa4', 64)]):
        p_in, p_out = pos_l[lvl], pos_l[lvl + 1]
        nn_idx, _ = _knn(p_out, p_in, 32)
        grouped_pos = p_in[nn_idx] - p_out[:, None, :]
        grouped = jnp.concatenate([grouped_pos, feats[-1][nn_idx]], axis=-1)
        feats.append(jnp.max(_mlp(grouped, params[name]), axis=1))

    f1, f2, f3, f4 = feats[1], feats[2], feats[3], feats[4]

    def fp(pos1, feat1, pos2, feat2, layers):
        idx, d = _knn(pos1, pos2, 3)
        w = 1.0 / (d + 1e-8)
        w = w / jnp.sum(w, axis=1, keepdims=True)
        interp = jnp.sum(_gather_rows(feat2, idx) * w[..., None], axis=1)
        x = interp if feat1 is None else jnp.concatenate([feat1, interp], axis=-1)
        return _mlp(x, layers)

    f3 = fp(pos_l[3], f3, pos_l[4], f4, params['fp4'])
    f2 = fp(pos_l[2], f2, pos_l[3], f3, params['fp3'])
    f1 = fp(pos_l[1], f1, pos_l[2], f2, params['fp2'])
    x = fp(pos_l[0], None, pos_l[1], f1, params['fp1'])
    (w1, b1), (w2, b2) = params['cls']
    x = jax.nn.relu(_bn(x @ w1 + b1))
    x = x @ w2 + b2
    return x
